# baseline (speedup 1.0000x reference)
import functools

import jax
import jax.numpy as jnp
from jax import lax
from jax.experimental import pallas as pl
from jax.experimental.pallas import tpu as pltpu
from jax.experimental.pallas import tpu_sc as plsc

H = 4
DH = 32
D = 128
N = 10000
NPAD = 10240
E = 320000
NF = NPAD * H

NC = 2
NS = 16
L = 16
NW = NC * NS

CHUNK = 128
WROWS = CHUNK * L // D
NCHUNKS = E // CHUNK
KMAX_A = (NCHUNKS + NW - 1) // NW

CHUNK_B = 256
SUBB = CHUNK_B // 128
WROWS_B = CHUNK_B * L // D
NCHUNKS_B = E // CHUNK_B
KMAX_B = (NCHUNKS_B + NS - 1) // NS

_mesh_a = plsc.VectorSubcoreMesh(core_axis_name="c", subcore_axis_name="s")
_mesh_b = plsc.VectorSubcoreMesh(core_axis_name="c", subcore_axis_name="s",
                                 num_cores=1)


def _proj_body(x_ref, w_ref, as_ref, ad_ref, h_ref, s_ref, d_ref, m_ref):
  x = x_ref[...]
  w = w_ref[...]
  h = lax.dot_general(x, w, (((1,), (1,)), ((), ())),
                      preferred_element_type=jnp.float32)
  h_ref[...] = h
  s = lax.dot_general(h, as_ref[...], (((1,), (0,)), ((), ())),
                      preferred_element_type=jnp.float32)
  d = lax.dot_general(h, ad_ref[...], (((1,), (0,)), ((), ())),
                      preferred_element_type=jnp.float32)
  s_ref[...] = s
  d_ref[...] = d
  smax = jnp.max(s, axis=0, keepdims=True)
  m_ref[...] = jnp.concatenate(
      [smax, jnp.full((1, L - H), 1e30, jnp.float32)], axis=1)


def _project(x_pad, w, amat_s, amat_d):
  return pl.pallas_call(
      _proj_body,
      out_shape=(
          jax.ShapeDtypeStruct((NPAD, D), jnp.float32),
          jax.ShapeDtypeStruct((NPAD, H), jnp.float32),
          jax.ShapeDtypeStruct((NPAD, H), jnp.float32),
          jax.ShapeDtypeStruct((1, L), jnp.float32),
      ),
  )(x_pad, w, amat_s, amat_d)


def _wz_body(src_hbm, dst_hbm, s_hbm, d_hbm, smax_hbm,
             w_hbm, zout_hbm,
             s_tab, d_tab, z_tab, esrc, edst, wbuf, smaxv, sem):
  cid_c = lax.axis_index("c")
  sid = lax.axis_index("s")
  wid = cid_c * NS + sid

  pltpu.sync_copy(s_hbm, s_tab)
  pltpu.sync_copy(d_hbm, d_tab)
  pltpu.sync_copy(smax_hbm, smaxv)

  def _zinit(i, _):
    z_tab[pl.ds(i * L, L)] = jnp.zeros((L,), jnp.float32)
    return 0
  lax.fori_loop(0, NF // L, _zinit, 0)

  smax_vec = smaxv[pl.ds(0, L)]
  lane_lt4 = lax.iota(jnp.int32, L) < H

  def _chunk(k, _):
    cid = wid + k * NW

    @pl.when(cid < NCHUNKS)
    def _():
      base = cid * CHUNK
      pltpu.sync_copy(src_hbm.at[pl.ds(base, CHUNK)], esrc.at[pl.ds(0, CHUNK)])
      pltpu.sync_copy(dst_hbm.at[pl.ds(base, CHUNK)], edst.at[pl.ds(0, CHUNK)])

      def _edge(r, _):
        ssc = esrc[pl.ds(r, L)][0]
        dsc = edst[pl.ds(r, L)][0]
        s4 = s_tab[pl.ds(ssc * H, L)]
        d4 = d_tab[pl.ds(dsc * H, L)]
        e4 = s4 + d4
        e4 = jnp.where(e4 > 0, e4, 0.2 * e4)
        c4 = smax_vec + d4
        c4 = jnp.where(c4 > 0, c4, 0.2 * c4)
        w4 = jnp.exp(e4 - c4)
        w4 = jnp.where(lane_lt4, w4, 0.0)
        zsl = pl.ds(dsc * H, L)
        z_tab[zsl] = z_tab[zsl] + w4
        wbuf[r >> 3, pl.ds((r & 7) * L, L)] = w4
        return 0
      lax.fori_loop(0, CHUNK, _edge, 0, unroll=4)

      pltpu.sync_copy(wbuf, w_hbm.at[pl.ds(cid * WROWS, WROWS)])
    return 0

  lax.fori_loop(0, KMAX_A, _chunk, 0)

  pltpu.sync_copy(z_tab, zout_hbm.at[pl.ds(wid * NF, NF)])


@functools.partial(
    pl.kernel,
    out_type=(
        jax.ShapeDtypeStruct((E * L // D, D), jnp.float32),
        jax.ShapeDtypeStruct((NW * NF,), jnp.float32),
    ),
    mesh=_mesh_a,
    scratch_types=dict(
        s_tab=pltpu.VMEM((NF,), jnp.float32),
        d_tab=pltpu.VMEM((NF,), jnp.float32),
        z_tab=pltpu.VMEM((NF,), jnp.float32),
        esrc=pltpu.VMEM((CHUNK + L,), jnp.int32),
        edst=pltpu.VMEM((CHUNK + L,), jnp.int32),
        wbuf=pltpu.VMEM((WROWS, D), jnp.float32),
        smaxv=pltpu.VMEM((L,), jnp.float32),
        sem=pltpu.SemaphoreType.DMA,
    ),
)
def _sc_wz_pass(src_hbm, dst_hbm, s_hbm, d_hbm, smax_hbm, w_hbm, zout_hbm,
                *, s_tab, d_tab, z_tab, esrc, edst, wbuf, smaxv, sem):
  _wz_body(src_hbm, dst_hbm, s_hbm, d_hbm, smax_hbm, w_hbm, zout_hbm,
           s_tab, d_tab, z_tab, esrc, edst, wbuf, smaxv, sem)


def _acc_body(src_hbm, dst_hbm, w_hbm, h_hbm, zeros_hbm,
              acc_hbm,
              acc_sp, esrc, edst2, hbuf, wbuf, sem):
  sid = lax.axis_index("s")

  r0 = sid * (NPAD // NS)
  pltpu.sync_copy(zeros_hbm.at[pl.ds(r0, NPAD // NS)],
                  acc_sp.at[pl.ds(r0, NPAD // NS)])
  plsc.subcore_barrier()

  def _chunk(k, _):
    cid = sid + k * NS

    @pl.when(cid < NCHUNKS_B)
    def _():
      base = cid * CHUNK_B
      cps = []
      for b in range(SUBB):
        cps.append(pltpu.async_copy(
            src_hbm.at[pl.ds(base + b * 128, 128)], esrc.at[b], sem))
        cps.append(pltpu.async_copy(
            dst_hbm.at[pl.ds(base + b * 128, 128)], edst2.at[b], sem))
      cps.append(pltpu.async_copy(
          w_hbm.at[pl.ds(cid * WROWS_B, WROWS_B)], wbuf, sem))
      for cp in cps[:-1]:
        cp.wait()
      cpg = [pltpu.async_copy(h_hbm.at[esrc.at[b]],
                              hbuf.at[pl.ds(b * 128, 128)], sem)
             for b in range(SUBB)]
      cps[-1].wait()
      for cp in cpg:
        cp.wait()

      def _edge(r, _):
        w4 = wbuf[r >> 3, pl.ds((r & 7) * L, L)]
        for q in range(D // L):
          sl = pl.ds(q * L, L)
          hbuf[r, sl] = hbuf[r, sl] * w4[q >> 1]
        return 0
      lax.fori_loop(0, CHUNK_B, _edge, 0, unroll=8)

      for b in range(SUBB):
        pltpu.sync_copy(hbuf.at[pl.ds(b * 128, 128)],
                        acc_sp.at[edst2.at[b]], add=True)
    return 0

  lax.fori_loop(0, KMAX_B, _chunk, 0)
  plsc.subcore_barrier()

  pltpu.sync_copy(acc_sp.at[pl.ds(r0, NPAD // NS)],
                  acc_hbm.at[pl.ds(r0, NPAD // NS)])


@functools.partial(
    pl.kernel,
    out_type=jax.ShapeDtypeStruct((NPAD, D), jnp.float32),
    mesh=_mesh_b,
    scratch_types=dict(
        acc_sp=pltpu.VMEM_SHARED((NPAD, D), jnp.float32),
        esrc=pltpu.VMEM((SUBB, 128), jnp.int32),
        edst2=pltpu.VMEM((SUBB, 128), jnp.int32),
        hbuf=pltpu.VMEM((CHUNK_B, D), jnp.float32),
        wbuf=pltpu.VMEM((WROWS_B, D), jnp.float32),
        sem=pltpu.SemaphoreType.DMA,
    ),
)
def _sc_acc_pass(src_hbm, dst_hbm, w_hbm, h_hbm, zeros_hbm, acc_hbm,
                 *, acc_sp, esrc, edst2, hbuf, wbuf, sem):
  _acc_body(src_hbm, dst_hbm, w_hbm, h_hbm, zeros_hbm, acc_hbm,
            acc_sp, esrc, edst2, hbuf, wbuf, sem)


def _zsum_body(z_ref, out_ref):
  out_ref[...] = jnp.sum(z_ref[...], axis=0)


def _zsum(zp2):
  return pl.pallas_call(
      _zsum_body,
      out_shape=jax.ShapeDtypeStruct((NF,), jnp.float32),
  )(zp2)


def _norm_body(acc_ref, z_ref, b_ref, out_ref):
  zinv = 1.0 / (z_ref[...] + 1e-12)
  z128 = lax.dot_general(zinv, b_ref[...], (((1,), (0,)), ((), ())),
                         preferred_element_type=jnp.float32)
  out_ref[...] = acc_ref[...] * z128


def _normalize(acc, z4, bmat):
  return pl.pallas_call(
      _norm_body,
      out_shape=jax.ShapeDtypeStruct((NPAD, D), jnp.float32),
  )(acc, z4, bmat)


def kernel(x, edge_index, num_nodes, W, a_src, a_dst):
  src = edge_index[0].astype(jnp.int32)
  dst = edge_index[1].astype(jnp.int32)
  x_pad = jnp.pad(x, ((0, NPAD - N), (0, 0)))
  eye = jnp.eye(H, dtype=jnp.float32)
  amat_s = (a_src[:, :, None] * eye[:, None, :]).reshape(H * DH, H)
  amat_d = (a_dst[:, :, None] * eye[:, None, :]).reshape(H * DH, H)

  h, s2, d2, smax = _project(x_pad, W, amat_s, amat_d)
  s_f = s2.reshape(NF)
  d_f = d2.reshape(NF)
  smax_f = smax.reshape(L)

  wpack, zout = _sc_wz_pass(src, dst, s_f, d_f, smax_f)
  zeros = jnp.zeros((NPAD, D), jnp.float32)
  acc = _sc_acc_pass(src, dst, wpack, h, zeros)
  z4 = _zsum(zout.reshape(NW, NF)).reshape(NPAD, H)
  bmat = jnp.repeat(eye, DH, axis=1)
  out = _normalize(acc, z4, bmat)
  return out[:N]


# speedup vs baseline: 1.0507x; 1.0507x over previous
import functools

import jax
import jax.numpy as jnp
from jax import lax
from jax.experimental import pallas as pl
from jax.experimental.pallas import tpu as pltpu
from jax.experimental.pallas import tpu_sc as plsc

H = 4
DH = 32
D = 128
N = 10000
NPAD = 10240
E = 320000
NF = NPAD * H

NC = 2
NS = 16
L = 16
NW = NC * NS

CHUNK = 128
WROWS = CHUNK * L // D
NCHUNKS = E // CHUNK
KMAX_A = (NCHUNKS + NW - 1) // NW

CHUNK_B = 128
WROWS_B = CHUNK_B * L // D
NCHUNKS_B = E // CHUNK_B
KMAX_B = (NCHUNKS_B + NS - 1) // NS

_mesh_a = plsc.VectorSubcoreMesh(core_axis_name="c", subcore_axis_name="s")
_mesh_b = plsc.VectorSubcoreMesh(core_axis_name="c", subcore_axis_name="s",
                                 num_cores=1)


def _proj_body(x_ref, w_ref, as_ref, ad_ref, h_ref, s_ref, d_ref, m_ref):
  x = x_ref[...]
  w = w_ref[...]
  h = lax.dot_general(x, w, (((1,), (1,)), ((), ())),
                      preferred_element_type=jnp.float32)
  h_ref[...] = h
  s = lax.dot_general(h, as_ref[...], (((1,), (0,)), ((), ())),
                      preferred_element_type=jnp.float32)
  d = lax.dot_general(h, ad_ref[...], (((1,), (0,)), ((), ())),
                      preferred_element_type=jnp.float32)
  s_ref[...] = s
  d_ref[...] = d
  smax = jnp.max(s, axis=0, keepdims=True)
  m_ref[...] = jnp.concatenate(
      [smax, jnp.full((1, L - H), 1e30, jnp.float32)], axis=1)


def _project(x_pad, w, amat_s, amat_d):
  return pl.pallas_call(
      _proj_body,
      out_shape=(
          jax.ShapeDtypeStruct((NPAD, D), jnp.float32),
          jax.ShapeDtypeStruct((NPAD, H), jnp.float32),
          jax.ShapeDtypeStruct((NPAD, H), jnp.float32),
          jax.ShapeDtypeStruct((1, L), jnp.float32),
      ),
  )(x_pad, w, amat_s, amat_d)


def _wz_body(src_hbm, dst_hbm, s_hbm, d_hbm, smax_hbm,
             w_hbm, zout_hbm,
             s_tab, d_tab, z_tab, esrc, edst, wbuf, smaxv, sem):
  cid_c = lax.axis_index("c")
  sid = lax.axis_index("s")
  wid = cid_c * NS + sid

  pltpu.sync_copy(s_hbm, s_tab)
  pltpu.sync_copy(d_hbm, d_tab)
  pltpu.sync_copy(smax_hbm, smaxv)

  def _zinit(i, _):
    z_tab[pl.ds(i * L, L)] = jnp.zeros((L,), jnp.float32)
    return 0
  lax.fori_loop(0, NF // L, _zinit, 0)

  smax_vec = smaxv[pl.ds(0, L)]
  lane_lt4 = lax.iota(jnp.int32, L) < H

  def _chunk(k, _):
    cid = wid + k * NW

    @pl.when(cid < NCHUNKS)
    def _():
      base = cid * CHUNK
      pltpu.sync_copy(src_hbm.at[pl.ds(base, CHUNK)], esrc.at[pl.ds(0, CHUNK)])
      pltpu.sync_copy(dst_hbm.at[pl.ds(base, CHUNK)], edst.at[pl.ds(0, CHUNK)])

      def _edge(r, _):
        ssc = esrc[pl.ds(r, L)][0]
        dsc = edst[pl.ds(r, L)][0]
        s4 = s_tab[pl.ds(ssc * H, L)]
        d4 = d_tab[pl.ds(dsc * H, L)]
        e4 = s4 + d4
        e4 = jnp.where(e4 > 0, e4, 0.2 * e4)
        c4 = smax_vec + d4
        c4 = jnp.where(c4 > 0, c4, 0.2 * c4)
        w4 = jnp.exp(e4 - c4)
        w4 = jnp.where(lane_lt4, w4, 0.0)
        zsl = pl.ds(dsc * H, L)
        z_tab[zsl] = z_tab[zsl] + w4
        wbuf[r >> 3, pl.ds((r & 7) * L, L)] = w4
        return 0
      lax.fori_loop(0, CHUNK, _edge, 0, unroll=4)

      pltpu.sync_copy(wbuf, w_hbm.at[pl.ds(cid * WROWS, WROWS)])
    return 0

  lax.fori_loop(0, KMAX_A, _chunk, 0)

  pltpu.sync_copy(z_tab, zout_hbm.at[pl.ds(wid * NF, NF)])


@functools.partial(
    pl.kernel,
    out_type=(
        jax.ShapeDtypeStruct((E * L // D, D), jnp.float32),
        jax.ShapeDtypeStruct((NW * NF,), jnp.float32),
    ),
    mesh=_mesh_a,
    scratch_types=dict(
        s_tab=pltpu.VMEM((NF,), jnp.float32),
        d_tab=pltpu.VMEM((NF,), jnp.float32),
        z_tab=pltpu.VMEM((NF,), jnp.float32),
        esrc=pltpu.VMEM((CHUNK + L,), jnp.int32),
        edst=pltpu.VMEM((CHUNK + L,), jnp.int32),
        wbuf=pltpu.VMEM((WROWS, D), jnp.float32),
        smaxv=pltpu.VMEM((L,), jnp.float32),
        sem=pltpu.SemaphoreType.DMA,
    ),
)
def _sc_wz_pass(src_hbm, dst_hbm, s_hbm, d_hbm, smax_hbm, w_hbm, zout_hbm,
                *, s_tab, d_tab, z_tab, esrc, edst, wbuf, smaxv, sem):
  _wz_body(src_hbm, dst_hbm, s_hbm, d_hbm, smax_hbm, w_hbm, zout_hbm,
           s_tab, d_tab, z_tab, esrc, edst, wbuf, smaxv, sem)


def _acc_body(src_hbm, dst_hbm, w_hbm, h_hbm, zeros_hbm,
              acc_hbm,
              acc_sp, esrc0, esrc1, edst2, hbuf0, hbuf1, wbuf0, wbuf1,
              semh0, semh1, semw0, semw1):
  sid = lax.axis_index("s")
  esrc_ = (esrc0, esrc1)
  hbuf_ = (hbuf0, hbuf1)
  wbuf_ = (wbuf0, wbuf1)
  semh_ = (semh0, semh1)
  semw_ = (semw0, semw1)

  r0 = sid * (NPAD // NS)
  pltpu.sync_copy(zeros_hbm.at[pl.ds(r0, NPAD // NS)],
                  acc_sp.at[pl.ds(r0, NPAD // NS)])
  plsc.subcore_barrier()

  def _fetch(k, b):
    cid = sid + k * NS

    @pl.when(cid < NCHUNKS_B)
    def _():
      base = cid * CHUNK_B
      pltpu.sync_copy(src_hbm.at[pl.ds(base, CHUNK_B)], esrc_[b])
      pltpu.sync_copy(dst_hbm.at[pl.ds(base, CHUNK_B)], edst2.at[b])
      pltpu.async_copy(w_hbm.at[pl.ds(cid * WROWS_B, WROWS_B)],
                       wbuf_[b], semw_[b])
      pltpu.async_copy(h_hbm.at[esrc_[b]], hbuf_[b], semh_[b])

  def _consume(k, b):
    cid = sid + k * NS
    hb = hbuf_[b]
    wb = wbuf_[b]

    @pl.when(cid < NCHUNKS_B)
    def _():
      pltpu.make_async_copy(w_hbm.at[pl.ds(cid * WROWS_B, WROWS_B)],
                            wb, semw_[b]).wait()
      pltpu.make_async_copy(h_hbm.at[esrc_[b]], hb, semh_[b]).wait()

      def _edge(r, _):
        w4 = wb[r >> 3, pl.ds((r & 7) * L, L)]
        for q in range(D // L):
          sl = pl.ds(q * L, L)
          hb[r, sl] = hb[r, sl] * w4[q >> 1]
        return 0
      lax.fori_loop(0, CHUNK_B, _edge, 0, unroll=8)

      pltpu.sync_copy(hb, acc_sp.at[edst2.at[b]], add=True)

  _fetch(0, 0)

  def _pair(kk, _):
    for b in range(2):
      k = kk * 2 + b
      _fetch(k + 1, 1 - b)
      _consume(k, b)
    return 0

  lax.fori_loop(0, (KMAX_B + 1) // 2, _pair, 0)
  plsc.subcore_barrier()

  pltpu.sync_copy(acc_sp.at[pl.ds(r0, NPAD // NS)],
                  acc_hbm.at[pl.ds(r0, NPAD // NS)])


@functools.partial(
    pl.kernel,
    out_type=jax.ShapeDtypeStruct((NPAD, D), jnp.float32),
    mesh=_mesh_b,
    scratch_types=dict(
        acc_sp=pltpu.VMEM_SHARED((NPAD, D), jnp.float32),
        esrc0=pltpu.VMEM((CHUNK_B,), jnp.int32),
        esrc1=pltpu.VMEM((CHUNK_B,), jnp.int32),
        edst2=pltpu.VMEM((2, CHUNK_B), jnp.int32),
        hbuf0=pltpu.VMEM((CHUNK_B, D), jnp.float32),
        hbuf1=pltpu.VMEM((CHUNK_B, D), jnp.float32),
        wbuf0=pltpu.VMEM((WROWS_B, D), jnp.float32),
        wbuf1=pltpu.VMEM((WROWS_B, D), jnp.float32),
        semh0=pltpu.SemaphoreType.DMA,
        semh1=pltpu.SemaphoreType.DMA,
        semw0=pltpu.SemaphoreType.DMA,
        semw1=pltpu.SemaphoreType.DMA,
    ),
)
def _sc_acc_pass(src_hbm, dst_hbm, w_hbm, h_hbm, zeros_hbm, acc_hbm,
                 *, acc_sp, esrc0, esrc1, edst2, hbuf0, hbuf1, wbuf0, wbuf1,
                 semh0, semh1, semw0, semw1):
  _acc_body(src_hbm, dst_hbm, w_hbm, h_hbm, zeros_hbm, acc_hbm,
            acc_sp, esrc0, esrc1, edst2, hbuf0, hbuf1, wbuf0, wbuf1,
            semh0, semh1, semw0, semw1)


def _zsum_body(z_ref, out_ref):
  out_ref[...] = jnp.sum(z_ref[...], axis=0)


def _zsum(zp2):
  return pl.pallas_call(
      _zsum_body,
      out_shape=jax.ShapeDtypeStruct((NF,), jnp.float32),
  )(zp2)


def _norm_body(acc_ref, z_ref, b_ref, out_ref):
  zinv = 1.0 / (z_ref[...] + 1e-12)
  z128 = lax.dot_general(zinv, b_ref[...], (((1,), (0,)), ((), ())),
                         preferred_element_type=jnp.float32)
  out_ref[...] = acc_ref[...] * z128


def _normalize(acc, z4, bmat):
  return pl.pallas_call(
      _norm_body,
      out_shape=jax.ShapeDtypeStruct((NPAD, D), jnp.float32),
  )(acc, z4, bmat)


def kernel(x, edge_index, num_nodes, W, a_src, a_dst):
  src = edge_index[0].astype(jnp.int32)
  dst = edge_index[1].astype(jnp.int32)
  x_pad = jnp.pad(x, ((0, NPAD - N), (0, 0)))
  eye = jnp.eye(H, dtype=jnp.float32)
  amat_s = (a_src[:, :, None] * eye[:, None, :]).reshape(H * DH, H)
  amat_d = (a_dst[:, :, None] * eye[:, None, :]).reshape(H * DH, H)

  h, s2, d2, smax = _project(x_pad, W, amat_s, amat_d)
  s_f = s2.reshape(NF)
  d_f = d2.reshape(NF)
  smax_f = smax.reshape(L)

  wpack, zout = _sc_wz_pass(src, dst, s_f, d_f, smax_f)
  zeros = jnp.zeros((NPAD, D), jnp.float32)
  acc = _sc_acc_pass(src, dst, wpack, h, zeros)
  z4 = _zsum(zout.reshape(NW, NF)).reshape(NPAD, H)
  bmat = jnp.repeat(eye, DH, axis=1)
  out = _normalize(acc, z4, bmat)
  return out[:N]


# speedup vs baseline: 1.1379x; 1.0830x over previous
import functools

import jax
import jax.numpy as jnp
from jax import lax
from jax.experimental import pallas as pl
from jax.experimental.pallas import tpu as pltpu
from jax.experimental.pallas import tpu_sc as plsc

H = 4
DH = 32
D = 128
N = 10000
NPAD = 10240
E = 320000
NF = NPAD * H

NC = 2
NS = 16
L = 16
NW = NC * NS

CHUNK = 128
WROWS = CHUNK * L // D
NCHUNKS = E // CHUNK
KMAX_A = (NCHUNKS + NW - 1) // NW

CHUNK_B = 128
WROWS_B = CHUNK_B * L // D
NCHUNKS_B = E // CHUNK_B
KMAX_B = (NCHUNKS_B + NS - 1) // NS

_mesh_a = plsc.VectorSubcoreMesh(core_axis_name="c", subcore_axis_name="s")
_mesh_b = plsc.VectorSubcoreMesh(core_axis_name="c", subcore_axis_name="s",
                                 num_cores=1)


def _proj_body(x_ref, w_ref, as_ref, ad_ref, h_ref, s_ref, d_ref, m_ref):
  x = x_ref[...]
  w = w_ref[...]
  h = lax.dot_general(x, w, (((1,), (1,)), ((), ())),
                      preferred_element_type=jnp.float32)
  h_ref[...] = h
  s = lax.dot_general(h, as_ref[...], (((1,), (0,)), ((), ())),
                      preferred_element_type=jnp.float32)
  d = lax.dot_general(h, ad_ref[...], (((1,), (0,)), ((), ())),
                      preferred_element_type=jnp.float32)
  s_ref[...] = s
  d_ref[...] = d
  smax = jnp.max(s, axis=0, keepdims=True)
  m_ref[...] = jnp.concatenate(
      [smax, jnp.full((1, L - H), 1e30, jnp.float32)], axis=1)


def _project(x_pad, w, amat_s, amat_d):
  return pl.pallas_call(
      _proj_body,
      out_shape=(
          jax.ShapeDtypeStruct((NPAD, D), jnp.float32),
          jax.ShapeDtypeStruct((NPAD, H), jnp.float32),
          jax.ShapeDtypeStruct((NPAD, H), jnp.float32),
          jax.ShapeDtypeStruct((1, L), jnp.float32),
      ),
  )(x_pad, w, amat_s, amat_d)


def _wz_body(src_hbm, dst_hbm, s_hbm, d_hbm, smax_hbm,
             w_hbm, zout_hbm,
             s_tab, d_tab, z_tab, esrc, edst, esrc1, edst1, wbuf, smaxv,
             semi0, semi1):
  cid_c = lax.axis_index("c")
  sid = lax.axis_index("s")
  wid = cid_c * NS + sid

  pltpu.sync_copy(s_hbm, s_tab)
  pltpu.sync_copy(d_hbm, d_tab)
  pltpu.sync_copy(smax_hbm, smaxv)

  def _zinit(i, _):
    z_tab[pl.ds(i * L, L)] = jnp.zeros((L,), jnp.float32)
    return 0
  lax.fori_loop(0, NF // L, _zinit, 0)

  smax_vec = smaxv[pl.ds(0, L)]
  lane_lt4 = lax.iota(jnp.int32, L) < H

  esrc_ = (esrc, esrc1)
  edst_ = (edst, edst1)
  semi_ = (semi0, semi1)

  def _fetch(k, b):
    cid = wid + k * NW

    @pl.when(cid < NCHUNKS)
    def _():
      base = cid * CHUNK
      pltpu.async_copy(src_hbm.at[pl.ds(base, CHUNK)],
                       esrc_[b].at[pl.ds(0, CHUNK)], semi_[b])
      pltpu.async_copy(dst_hbm.at[pl.ds(base, CHUNK)],
                       edst_[b].at[pl.ds(0, CHUNK)], semi_[b])

  def _consume(k, b):
    cid = wid + k * NW
    es = esrc_[b]
    ed = edst_[b]

    @pl.when(cid < NCHUNKS)
    def _():
      base = cid * CHUNK
      pltpu.make_async_copy(src_hbm.at[pl.ds(base, CHUNK)],
                            es.at[pl.ds(0, CHUNK)], semi_[b]).wait()
      pltpu.make_async_copy(dst_hbm.at[pl.ds(base, CHUNK)],
                            ed.at[pl.ds(0, CHUNK)], semi_[b]).wait()

      def _edge(r, _):
        ssc = es[pl.ds(r, L)][0]
        dsc = ed[pl.ds(r, L)][0]
        s4 = s_tab[pl.ds(ssc * H, L)]
        d4 = d_tab[pl.ds(dsc * H, L)]
        e4 = s4 + d4
        e4 = jnp.where(e4 > 0, e4, 0.2 * e4)
        c4 = smax_vec + d4
        c4 = jnp.where(c4 > 0, c4, 0.2 * c4)
        w4 = jnp.exp(e4 - c4)
        w4 = jnp.where(lane_lt4, w4, 0.0)
        zsl = pl.ds(dsc * H, L)
        z_tab[zsl] = z_tab[zsl] + w4
        wbuf[r >> 3, pl.ds((r & 7) * L, L)] = w4
        return 0
      lax.fori_loop(0, CHUNK, _edge, 0, unroll=4)

      pltpu.sync_copy(wbuf, w_hbm.at[pl.ds(cid * WROWS, WROWS)])

  _fetch(0, 0)

  def _pair(kk, _):
    for b in range(2):
      k = kk * 2 + b
      _fetch(k + 1, 1 - b)
      _consume(k, b)
    return 0

  lax.fori_loop(0, (KMAX_A + 1) // 2, _pair, 0)

  pltpu.sync_copy(z_tab, zout_hbm.at[pl.ds(wid * NF, NF)])


@functools.partial(
    pl.kernel,
    out_type=(
        jax.ShapeDtypeStruct((E * L // D, D), jnp.float32),
        jax.ShapeDtypeStruct((NW * NF,), jnp.float32),
    ),
    mesh=_mesh_a,
    scratch_types=dict(
        s_tab=pltpu.VMEM((NF,), jnp.float32),
        d_tab=pltpu.VMEM((NF,), jnp.float32),
        z_tab=pltpu.VMEM((NF,), jnp.float32),
        esrc=pltpu.VMEM((CHUNK + L,), jnp.int32),
        edst=pltpu.VMEM((CHUNK + L,), jnp.int32),
        esrc1=pltpu.VMEM((CHUNK + L,), jnp.int32),
        edst1=pltpu.VMEM((CHUNK + L,), jnp.int32),
        wbuf=pltpu.VMEM((WROWS, D), jnp.float32),
        smaxv=pltpu.VMEM((L,), jnp.float32),
        semi0=pltpu.SemaphoreType.DMA,
        semi1=pltpu.SemaphoreType.DMA,
    ),
)
def _sc_wz_pass(src_hbm, dst_hbm, s_hbm, d_hbm, smax_hbm, w_hbm, zout_hbm,
                *, s_tab, d_tab, z_tab, esrc, edst, esrc1, edst1, wbuf,
                smaxv, semi0, semi1):
  _wz_body(src_hbm, dst_hbm, s_hbm, d_hbm, smax_hbm, w_hbm, zout_hbm,
           s_tab, d_tab, z_tab, esrc, edst, esrc1, edst1, wbuf, smaxv,
           semi0, semi1)


def _acc_body(src_hbm, dst_hbm, w_hbm, h_hbm, zeros_hbm,
              acc_hbm,
              acc_sp, esrc0, esrc1, edst2, hbuf0, hbuf1, wbuf0, wbuf1,
              semh0, semh1, semw0, semw1):
  sid = lax.axis_index("s")
  esrc_ = (esrc0, esrc1)
  hbuf_ = (hbuf0, hbuf1)
  wbuf_ = (wbuf0, wbuf1)
  semh_ = (semh0, semh1)
  semw_ = (semw0, semw1)

  r0 = sid * (NPAD // NS)
  pltpu.sync_copy(zeros_hbm.at[pl.ds(r0, NPAD // NS)],
                  acc_sp.at[pl.ds(r0, NPAD // NS)])
  plsc.subcore_barrier()

  def _fetch(k, b):
    cid = sid + k * NS

    @pl.when(cid < NCHUNKS_B)
    def _():
      base = cid * CHUNK_B
      pltpu.sync_copy(src_hbm.at[pl.ds(base, CHUNK_B)], esrc_[b])
      pltpu.sync_copy(dst_hbm.at[pl.ds(base, CHUNK_B)], edst2.at[b])
      pltpu.async_copy(w_hbm.at[pl.ds(cid * WROWS_B, WROWS_B)],
                       wbuf_[b], semw_[b])
      pltpu.async_copy(h_hbm.at[esrc_[b]], hbuf_[b], semh_[b])

  def _consume(k, b):
    cid = sid + k * NS
    hb = hbuf_[b]
    wb = wbuf_[b]

    @pl.when(cid < NCHUNKS_B)
    def _():
      pltpu.make_async_copy(w_hbm.at[pl.ds(cid * WROWS_B, WROWS_B)],
                            wb, semw_[b]).wait()
      pltpu.make_async_copy(h_hbm.at[esrc_[b]], hb, semh_[b]).wait()

      def _edge(r, _):
        w4 = wb[r >> 3, pl.ds((r & 7) * L, L)]
        for q in range(D // L):
          sl = pl.ds(q * L, L)
          hb[r, sl] = hb[r, sl] * w4[q >> 1]
        return 0
      lax.fori_loop(0, CHUNK_B, _edge, 0, unroll=8)

      pltpu.sync_copy(hb, acc_sp.at[edst2.at[b]], add=True)

  _fetch(0, 0)

  def _pair(kk, _):
    for b in range(2):
      k = kk * 2 + b
      _fetch(k + 1, 1 - b)
      _consume(k, b)
    return 0

  lax.fori_loop(0, (KMAX_B + 1) // 2, _pair, 0)
  plsc.subcore_barrier()

  pltpu.sync_copy(acc_sp.at[pl.ds(r0, NPAD // NS)],
                  acc_hbm.at[pl.ds(r0, NPAD // NS)])


@functools.partial(
    pl.kernel,
    out_type=jax.ShapeDtypeStruct((NPAD, D), jnp.float32),
    mesh=_mesh_b,
    scratch_types=dict(
        acc_sp=pltpu.VMEM_SHARED((NPAD, D), jnp.float32),
        esrc0=pltpu.VMEM((CHUNK_B,), jnp.int32),
        esrc1=pltpu.VMEM((CHUNK_B,), jnp.int32),
        edst2=pltpu.VMEM((2, CHUNK_B), jnp.int32),
        hbuf0=pltpu.VMEM((CHUNK_B, D), jnp.float32),
        hbuf1=pltpu.VMEM((CHUNK_B, D), jnp.float32),
        wbuf0=pltpu.VMEM((WROWS_B, D), jnp.float32),
        wbuf1=pltpu.VMEM((WROWS_B, D), jnp.float32),
        semh0=pltpu.SemaphoreType.DMA,
        semh1=pltpu.SemaphoreType.DMA,
        semw0=pltpu.SemaphoreType.DMA,
        semw1=pltpu.SemaphoreType.DMA,
    ),
)
def _sc_acc_pass(src_hbm, dst_hbm, w_hbm, h_hbm, zeros_hbm, acc_hbm,
                 *, acc_sp, esrc0, esrc1, edst2, hbuf0, hbuf1, wbuf0, wbuf1,
                 semh0, semh1, semw0, semw1):
  _acc_body(src_hbm, dst_hbm, w_hbm, h_hbm, zeros_hbm, acc_hbm,
            acc_sp, esrc0, esrc1, edst2, hbuf0, hbuf1, wbuf0, wbuf1,
            semh0, semh1, semw0, semw1)


def _zsum_body(z_ref, out_ref):
  out_ref[...] = jnp.sum(z_ref[...], axis=0)


def _zsum(zp2):
  return pl.pallas_call(
      _zsum_body,
      out_shape=jax.ShapeDtypeStruct((NF,), jnp.float32),
  )(zp2)


def _norm_body(acc_ref, z_ref, b_ref, out_ref):
  zinv = 1.0 / (z_ref[...] + 1e-12)
  z128 = lax.dot_general(zinv, b_ref[...], (((1,), (0,)), ((), ())),
                         preferred_element_type=jnp.float32)
  out_ref[...] = acc_ref[...] * z128


def _normalize(acc, z4, bmat):
  return pl.pallas_call(
      _norm_body,
      out_shape=jax.ShapeDtypeStruct((NPAD, D), jnp.float32),
  )(acc, z4, bmat)


def kernel(x, edge_index, num_nodes, W, a_src, a_dst):
  src = edge_index[0].astype(jnp.int32)
  dst = edge_index[1].astype(jnp.int32)
  x_pad = jnp.pad(x, ((0, NPAD - N), (0, 0)))
  eye = jnp.eye(H, dtype=jnp.float32)
  amat_s = (a_src[:, :, None] * eye[:, None, :]).reshape(H * DH, H)
  amat_d = (a_dst[:, :, None] * eye[:, None, :]).reshape(H * DH, H)

  h, s2, d2, smax = _project(x_pad, W, amat_s, amat_d)
  s_f = s2.reshape(NF)
  d_f = d2.reshape(NF)
  smax_f = smax.reshape(L)

  wpack, zout = _sc_wz_pass(src, dst, s_f, d_f, smax_f)
  zeros = jnp.zeros((NPAD, D), jnp.float32)
  acc = _sc_acc_pass(src, dst, wpack, h, zeros)
  z4 = _zsum(zout.reshape(NW, NF)).reshape(NPAD, H)
  bmat = jnp.repeat(eye, DH, axis=1)
  out = _normalize(acc, z4, bmat)
  return out[:N]
